# baseline (speedup 1.0000x reference)
import functools

import numpy as np
import jax
import jax.numpy as jnp
from jax import lax
from jax.experimental import pallas as pl
from jax.experimental.pallas import tpu as pltpu
from jax.experimental.pallas import tpu_sc as plsc

IMG_H, IMG_W = 512, 512
FH, FW = 32, 32
C_IN, HID = 384, 512
A = 9
L = 8
HW = FH * FW
N_ANC = HW * A
PRE_NMS, POST_NMS = 512, 128
MIN_SCORE, IOU_THR, MIN_SIZE = 0.5, 0.7, 1e-3
ROWS_PER_IMG = 520
N_TILES = 32
ELEM_PER_TILE = L * N_ANC // N_TILES
SCAT_BATCH = 128


def _anchor_consts():
    sy, sx = IMG_H / FH, IMG_W / FW
    fw = np.arange(FW, dtype=np.float32)
    fh = np.arange(FH, dtype=np.float32)
    cx = (fw + 0.5) * np.float32(sx)
    cy = (fh + 0.5) * np.float32(sy)
    wh = np.array([[s / np.sqrt(r), s * np.sqrt(r)]
                   for s in (32.0, 64.0, 128.0) for r in (0.5, 1.0, 2.0)],
                  dtype=np.float32)
    ACX = np.broadcast_to(np.tile(cx, FH)[None, :], (A, HW)).copy()
    ACY = np.broadcast_to(np.repeat(cy, FW)[None, :], (A, HW)).copy()
    AW = np.broadcast_to(wh[:, 0][:, None], (A, HW)).copy()
    AH = np.broadcast_to(wh[:, 1][:, None], (A, HW)).copy()
    IREF = (np.arange(HW, dtype=np.float32)[None, :] * A
            + np.arange(A, dtype=np.float32)[:, None])
    flat = np.arange(A * HW).reshape(A, HW)
    DUMP = (512 + flat // (N_ANC // 4)).astype(np.int32)
    return ACX, ACY, AW, AH, IREF, DUMP


def _prefix_lanes(y):
    n = y.shape[1]
    k = 1
    while k < n:
        pad = jnp.zeros((y.shape[0], k), y.dtype)
        y = y + jnp.concatenate([pad, y[:, :-k]], axis=1)
        k *= 2
    return y


def _prefix_rows(y):
    n = y.shape[0]
    k = 1
    while k < n:
        pad = jnp.zeros((k, y.shape[1]), y.dtype)
        y = y + jnp.concatenate([pad, y[:-k, :]], axis=0)
        k *= 2
    return y


def _dot_hi(a, b):
    return lax.dot_general(a, b, (((1,), (0,)), ((), ())),
                           precision=lax.Precision.HIGHEST,
                           preferred_element_type=jnp.float32)


def _head_body(h_ref, w2_ref, b2_ref, w3_ref, b3_ref,
               acx_ref, acy_ref, aw_ref, ah_ref, dump_ref, iref_ref,
               triu_ref, tri9_ref, v16_ref, dest_ref):
    h = h_ref[0]

    logit = lax.dot_general(w2_ref[...], h, (((1,), (0,)), ((), ())),
                            preferred_element_type=jnp.float32)
    logit = logit + b2_ref[:, 0][:, None]
    sc = 1.0 / (1.0 + jnp.exp(-logit))
    off = lax.dot_general(w3_ref[...], h, (((1,), (0,)), ((), ())),
                          preferred_element_type=jnp.float32)
    off = off + b3_ref[:, 0][:, None]

    dxo, dyo, dwo, dho = off[0:9], off[9:18], off[18:27], off[27:36]
    acx, acy = acx_ref[...], acy_ref[...]
    aw, ah = aw_ref[...], ah_ref[...]
    cx = acx + dxo * aw
    cy = acy + dyo * ah
    w = aw * jnp.exp(jnp.clip(dwo, -10.0, 10.0))
    hh = ah * jnp.exp(jnp.clip(dho, -10.0, 10.0))
    x1 = jnp.clip(cx - w / 2, 0.0, float(IMG_W))
    y1 = jnp.clip(cy - hh / 2, 0.0, float(IMG_H))
    x2 = jnp.clip(cx + w / 2, 0.0, float(IMG_W))
    y2 = jnp.clip(cy + hh / 2, 0.0, float(IMG_H))

    sbits = lax.bitcast_convert_type(sc, jnp.int32)

    def bis_body(_, lh):
        lo, hi = lh
        mid = lo + ((hi - lo) >> 1)
        cnt = jnp.sum((sbits > mid).astype(jnp.int32))
        return jnp.where(cnt < PRE_NMS, lo, mid + 1), jnp.where(cnt < PRE_NMS, mid, hi)

    lo, _ = lax.fori_loop(0, 31, bis_body, (jnp.int32(0), jnp.int32(2**31 - 1)))
    T = lo
    gt = (sbits > T).astype(jnp.int32)
    eq = (sbits == T).astype(jnp.int32)
    need = PRE_NMS - jnp.sum(gt)

    triu = triu_ref[...]
    tri9 = tri9_ref[...]

    def excl_prefix(m):
        mf = m.astype(jnp.float32)
        incl = lax.dot_general(mf, triu, (((1,), (0,)), ((), ())),
                               preferred_element_type=jnp.float32)
        rowtot = incl[:, HW - 1:HW]
        roff = _dot_hi(tri9, rowtot)
        return (incl + roff - mf).astype(jnp.int32)

    sel = (gt + eq * (excl_prefix(eq) < need).astype(jnp.int32)) > 0
    dest_local = excl_prefix(sel.astype(jnp.int32))
    base = pl.program_id(0) * ROWS_PER_IMG
    dest = jnp.where(sel, dest_local, dump_ref[...]) + base

    pad2 = jnp.zeros((2, HW), jnp.float32)
    iref = iref_ref[...]
    for a_ in range(A):
        S = jnp.concatenate(
            [sc[a_:a_ + 1], x1[a_:a_ + 1], y1[a_:a_ + 1], x2[a_:a_ + 1],
             y2[a_:a_ + 1], iref[a_:a_ + 1], pad2], axis=0)
        v16_ref[0, a_ * HW:(a_ + 1) * HW, :] = jnp.transpose(S)
    dest_ref[0] = dest


def _sc_route_body(vals_hbm, idx_hbm, out_hbm, rows_v, idx_v, sem):
    wid = lax.axis_index("s") * 2 + lax.axis_index("c")
    pltpu.sync_copy(vals_hbm.at[wid], rows_v)
    pltpu.sync_copy(idx_hbm.at[wid], idx_v)
    handles = []
    for j in range(ELEM_PER_TILE // SCAT_BATCH):
        handles.append(pltpu.async_copy(
            rows_v.at[pl.ds(j * SCAT_BATCH, SCAT_BATCH)],
            out_hbm.at[idx_v.at[j]], sem))
    for hcopy in handles:
        hcopy.wait()


def _tail_body(comp_ref, out_ref):
    comp = comp_ref[0, 0:PRE_NMS, :]
    scs = comp[:, 0:1]
    irs = comp[:, 5:6]
    scT = jnp.transpose(scs)
    irT = jnp.transpose(irs)
    before = (scT > scs) | ((scT == scs) & (irT < irs))
    rank = jnp.sum(before.astype(jnp.int32), axis=1, keepdims=True)
    iota_r = lax.broadcasted_iota(jnp.int32, (PRE_NMS, PRE_NMS), 0)
    P = (jnp.transpose(rank) == iota_r).astype(jnp.float32)
    srt = _dot_hi(P, comp)

    ssc = srt[:, 0]
    sx1, sy1, sx2, sy2 = srt[:, 1], srt[:, 2], srt[:, 3], srt[:, 4]
    valid = (((sx2 - sx1) >= MIN_SIZE) & ((sy2 - sy1) >= MIN_SIZE)
             & (ssc >= MIN_SCORE))
    area = jnp.maximum(sx2 - sx1, 0.0) * jnp.maximum(sy2 - sy1, 0.0)
    ix1 = jnp.maximum(sx1[:, None], sx1[None, :])
    iy1 = jnp.maximum(sy1[:, None], sy1[None, :])
    ix2 = jnp.minimum(sx2[:, None], sx2[None, :])
    iy2 = jnp.minimum(sy2[:, None], sy2[None, :])
    inter = jnp.maximum(ix2 - ix1, 0.0) * jnp.maximum(iy2 - iy1, 0.0)
    iou = inter / (area[:, None] + area[None, :] - inter + 1e-9)

    rowi = lax.broadcasted_iota(jnp.int32, (PRE_NMS, PRE_NMS), 0)
    coli = lax.broadcasted_iota(jnp.int32, (PRE_NMS, PRE_NMS), 1)
    Mf = ((iou > IOU_THR) & (rowi < coli)).astype(jnp.float32)

    validrow = valid[None, :].astype(jnp.float32)

    def nms_cond(c):
        a_cur, a_prev, it = c
        return jnp.logical_and(jnp.any(a_cur != a_prev), it < PRE_NMS + 2)

    def nms_body(c):
        a_cur, _, it = c
        s = lax.dot_general(a_cur, Mf, (((1,), (0,)), ((), ())),
                            preferred_element_type=jnp.float32)
        return jnp.where(s > 0.0, 0.0, validrow), a_cur, it + 1

    act, _, _ = lax.while_loop(nms_cond, nms_body,
                               (validrow, validrow - 1.0, jnp.int32(0)))

    acti = act.astype(jnp.int32)
    kdest = _prefix_lanes(acti) - 1
    iota128 = lax.broadcasted_iota(jnp.int32, (POST_NMS, PRE_NMS), 0)
    oh2 = ((kdest == iota128) & (acti > 0)).astype(jnp.float32)
    out_ref[0] = _dot_hi(oh2, srt)


def _const_spec(shape):
    nd = len(shape)
    return pl.BlockSpec(shape, lambda i: (0,) * nd)


def kernel(feature_maps, W1, b1, W2, b2, W3, b3):
    x = feature_maps.reshape((-1, C_IN, FH, FW))
    y = lax.conv_general_dilated(x, W1, (1, 1), [(1, 1), (1, 1)],
                                 dimension_numbers=('NCHW', 'OIHW', 'NCHW'))
    h = jax.nn.relu(y + b1[None, :, None, None]).reshape(L, HID, HW)

    ACX, ACY, AW, AH, IREF, DUMP = _anchor_consts()
    perm = np.array([4 * a_ + k for k in range(4) for a_ in range(A)])
    W2r = W2.reshape(A, HID)
    W3g = W3.reshape(4 * A, HID)[perm]
    b3g = b3[perm]

    head_out_shapes = [jax.ShapeDtypeStruct((L, N_ANC, 8), jnp.float32),
                       jax.ShapeDtypeStruct((L, A, HW), jnp.int32)]
    v16, dest_a = pl.pallas_call(
        _head_body,
        grid=(L,),
        in_specs=[
            pl.BlockSpec((1, HID, HW), lambda i: (i, 0, 0)),
            _const_spec((A, HID)),
            _const_spec((A, 1)),
            _const_spec((4 * A, HID)),
            _const_spec((4 * A, 1)),
            _const_spec((A, HW)),
            _const_spec((A, HW)),
            _const_spec((A, HW)),
            _const_spec((A, HW)),
            _const_spec((A, HW)),
            _const_spec((A, HW)),
            _const_spec((HW, HW)),
            _const_spec((A, A)),
        ],
        out_specs=[pl.BlockSpec((1, N_ANC, 8), lambda i: (i, 0, 0)),
                   pl.BlockSpec((1, A, HW), lambda i: (i, 0, 0))],
        out_shape=head_out_shapes,
        compiler_params=pltpu.CompilerParams(
            dimension_semantics=("parallel",)),
    )(h, W2r, b2[:, None], W3g, b3g[:, None],
      jnp.asarray(ACX), jnp.asarray(ACY), jnp.asarray(AW), jnp.asarray(AH),
      jnp.asarray(DUMP), jnp.asarray(IREF),
      jnp.asarray(np.triu(np.ones((HW, HW), np.float32))),
      jnp.asarray(np.tril(np.ones((A, A), np.float32), -1)))

    vals_t = v16.reshape(N_TILES, ELEM_PER_TILE, 8)
    idx_t = dest_a.reshape(N_TILES, ELEM_PER_TILE // SCAT_BATCH, SCAT_BATCH)

    mesh = plsc.VectorSubcoreMesh(core_axis_name="c", subcore_axis_name="s")
    comp = functools.partial(
        pl.kernel, mesh=mesh,
        compiler_params=pltpu.CompilerParams(use_tc_tiling_on_sc=False),
        out_type=jax.ShapeDtypeStruct((L * ROWS_PER_IMG, 8), jnp.float32),
        scratch_types=[
            pltpu.VMEM((ELEM_PER_TILE, 8), jnp.float32),
            pltpu.VMEM((ELEM_PER_TILE // SCAT_BATCH, SCAT_BATCH), jnp.int32),
            pltpu.SemaphoreType.DMA,
        ],
    )(_sc_route_body)(vals_t, idx_t)

    comp = comp.reshape(L, ROWS_PER_IMG, 8)

    out = pl.pallas_call(
        _tail_body,
        grid=(L,),
        in_specs=[pl.BlockSpec((1, ROWS_PER_IMG, 8), lambda i: (i, 0, 0))],
        out_specs=pl.BlockSpec((1, POST_NMS, 8), lambda i: (i, 0, 0)),
        out_shape=jax.ShapeDtypeStruct((L, POST_NMS, 8), jnp.float32),
        compiler_params=pltpu.CompilerParams(
            dimension_semantics=("parallel",)),
    )(comp)

    return out[:, :, 1:5], out[:, :, 0]


# speedup vs baseline: 1.0251x; 1.0251x over previous
import functools

import numpy as np
import jax
import jax.numpy as jnp
from jax import lax
from jax.experimental import pallas as pl
from jax.experimental.pallas import tpu as pltpu
from jax.experimental.pallas import tpu_sc as plsc

IMG_H, IMG_W = 512, 512
FH, FW = 32, 32
C_IN, HID = 384, 512
A = 9
L = 8
HW = FH * FW
N_ANC = HW * A
PRE_NMS, POST_NMS = 512, 128
MIN_SCORE, IOU_THR, MIN_SIZE = 0.5, 0.7, 1e-3
ROWS_PER_IMG = 520
N_TILES = 32
ELEM_PER_TILE = L * N_ANC // N_TILES
SCAT_BATCH = 128


def _anchor_consts():
    sy, sx = IMG_H / FH, IMG_W / FW
    fw = np.arange(FW, dtype=np.float32)
    fh = np.arange(FH, dtype=np.float32)
    cx = (fw + 0.5) * np.float32(sx)
    cy = (fh + 0.5) * np.float32(sy)
    wh = np.array([[s / np.sqrt(r), s * np.sqrt(r)]
                   for s in (32.0, 64.0, 128.0) for r in (0.5, 1.0, 2.0)],
                  dtype=np.float32)
    ACX = np.broadcast_to(np.tile(cx, FH)[None, :], (A, HW)).copy()
    ACY = np.broadcast_to(np.repeat(cy, FW)[None, :], (A, HW)).copy()
    AW = np.broadcast_to(wh[:, 0][:, None], (A, HW)).copy()
    AH = np.broadcast_to(wh[:, 1][:, None], (A, HW)).copy()
    IREF = (np.arange(HW, dtype=np.float32)[None, :] * A
            + np.arange(A, dtype=np.float32)[:, None])
    flat = np.arange(A * HW).reshape(A, HW)
    DUMP = (512 + flat // (N_ANC // 4)).astype(np.int32)
    return ACX, ACY, AW, AH, IREF, DUMP


def _prefix_lanes(y):
    n = y.shape[1]
    k = 1
    while k < n:
        pad = jnp.zeros((y.shape[0], k), y.dtype)
        y = y + jnp.concatenate([pad, y[:, :-k]], axis=1)
        k *= 2
    return y


def _prefix_rows(y):
    n = y.shape[0]
    k = 1
    while k < n:
        pad = jnp.zeros((k, y.shape[1]), y.dtype)
        y = y + jnp.concatenate([pad, y[:-k, :]], axis=0)
        k *= 2
    return y


def _dot_hi(a, b):
    return lax.dot_general(a, b, (((1,), (0,)), ((), ())),
                           precision=lax.Precision.HIGHEST,
                           preferred_element_type=jnp.float32)


def _head_body(h_ref, w2_ref, b2_ref, w3_ref, b3_ref,
               acx_ref, acy_ref, aw_ref, ah_ref, dump_ref, iref_ref,
               triu_ref, tri9_ref, v16_ref, dest_ref):
    h = h_ref[0]

    logit = lax.dot_general(w2_ref[...], h, (((1,), (0,)), ((), ())),
                            preferred_element_type=jnp.float32)
    logit = logit + b2_ref[:, 0][:, None]
    sc = 1.0 / (1.0 + jnp.exp(-logit))
    off = lax.dot_general(w3_ref[...], h, (((1,), (0,)), ((), ())),
                          preferred_element_type=jnp.float32)
    off = off + b3_ref[:, 0][:, None]

    dxo, dyo, dwo, dho = off[0:9], off[9:18], off[18:27], off[27:36]
    acx, acy = acx_ref[...], acy_ref[...]
    aw, ah = aw_ref[...], ah_ref[...]
    cx = acx + dxo * aw
    cy = acy + dyo * ah
    w = aw * jnp.exp(jnp.clip(dwo, -10.0, 10.0))
    hh = ah * jnp.exp(jnp.clip(dho, -10.0, 10.0))
    x1 = jnp.clip(cx - w / 2, 0.0, float(IMG_W))
    y1 = jnp.clip(cy - hh / 2, 0.0, float(IMG_H))
    x2 = jnp.clip(cx + w / 2, 0.0, float(IMG_W))
    y2 = jnp.clip(cy + hh / 2, 0.0, float(IMG_H))

    sbits = lax.bitcast_convert_type(sc, jnp.int32)

    def bis_body(_, lh):
        lo, hi = lh
        mid = lo + ((hi - lo) >> 1)
        cnt = jnp.sum((sbits > mid).astype(jnp.int32))
        return jnp.where(cnt < PRE_NMS, lo, mid + 1), jnp.where(cnt < PRE_NMS, mid, hi)

    lo, _ = lax.fori_loop(0, 31, bis_body, (jnp.int32(0), jnp.int32(2**31 - 1)))
    T = lo
    gt = (sbits > T).astype(jnp.int32)
    eq = (sbits == T).astype(jnp.int32)
    need = PRE_NMS - jnp.sum(gt)

    triu = triu_ref[...]
    tri9 = tri9_ref[...]

    def excl_prefix(m):
        mf = m.astype(jnp.float32)
        incl = lax.dot_general(mf, triu, (((1,), (0,)), ((), ())),
                               preferred_element_type=jnp.float32)
        rowtot = incl[:, HW - 1:HW]
        roff = _dot_hi(tri9, rowtot)
        return (incl + roff - mf).astype(jnp.int32)

    sel = (gt + eq * (excl_prefix(eq) < need).astype(jnp.int32)) > 0
    dest_local = excl_prefix(sel.astype(jnp.int32))
    base = pl.program_id(0) * ROWS_PER_IMG
    dest = jnp.where(sel, dest_local, dump_ref[...]) + base

    pad10 = jnp.zeros((10, HW), jnp.float32)
    iref = iref_ref[...]
    for a_ in range(A):
        S = jnp.concatenate(
            [sc[a_:a_ + 1], x1[a_:a_ + 1], y1[a_:a_ + 1], x2[a_:a_ + 1],
             y2[a_:a_ + 1], iref[a_:a_ + 1], pad10], axis=0)
        v16_ref[0, a_ * HW:(a_ + 1) * HW, :] = jnp.transpose(S)
    dest_ref[0] = dest


def _sc_route_body(vals_hbm, idx_hbm, out_hbm, rows_v, idx_v, sem):
    wid = lax.axis_index("s") * 2 + lax.axis_index("c")
    pltpu.sync_copy(vals_hbm.at[wid], rows_v)
    pltpu.sync_copy(idx_hbm.at[wid], idx_v)
    handles = []
    for j in range(ELEM_PER_TILE // SCAT_BATCH):
        handles.append(pltpu.async_copy(
            rows_v.at[pl.ds(j * SCAT_BATCH, SCAT_BATCH)],
            out_hbm.at[idx_v.at[j]], sem))
    for hcopy in handles:
        hcopy.wait()


def _tail_body(comp_ref, out_ref):
    comp = comp_ref[0, 0:PRE_NMS, :]
    scs = comp[:, 0:1]
    irs = comp[:, 5:6]
    scT = jnp.transpose(scs)
    irT = jnp.transpose(irs)
    before = (scT > scs) | ((scT == scs) & (irT < irs))
    rank = jnp.sum(before.astype(jnp.int32), axis=1, keepdims=True)
    iota_r = lax.broadcasted_iota(jnp.int32, (PRE_NMS, PRE_NMS), 0)
    P = (jnp.transpose(rank) == iota_r).astype(jnp.float32)
    srt = _dot_hi(P, comp)

    ssc = srt[:, 0]
    sx1, sy1, sx2, sy2 = srt[:, 1], srt[:, 2], srt[:, 3], srt[:, 4]
    valid = (((sx2 - sx1) >= MIN_SIZE) & ((sy2 - sy1) >= MIN_SIZE)
             & (ssc >= MIN_SCORE))
    area = jnp.maximum(sx2 - sx1, 0.0) * jnp.maximum(sy2 - sy1, 0.0)
    ix1 = jnp.maximum(sx1[:, None], sx1[None, :])
    iy1 = jnp.maximum(sy1[:, None], sy1[None, :])
    ix2 = jnp.minimum(sx2[:, None], sx2[None, :])
    iy2 = jnp.minimum(sy2[:, None], sy2[None, :])
    inter = jnp.maximum(ix2 - ix1, 0.0) * jnp.maximum(iy2 - iy1, 0.0)
    iou = inter / (area[:, None] + area[None, :] - inter + 1e-9)

    rowi = lax.broadcasted_iota(jnp.int32, (PRE_NMS, PRE_NMS), 0)
    coli = lax.broadcasted_iota(jnp.int32, (PRE_NMS, PRE_NMS), 1)
    Mf = ((iou > IOU_THR) & (rowi < coli)).astype(jnp.float32)

    validrow = valid[None, :].astype(jnp.float32)

    def nms_cond(c):
        a_cur, a_prev, it = c
        return jnp.logical_and(jnp.any(a_cur != a_prev), it < PRE_NMS + 2)

    def nms_body(c):
        a_cur, _, it = c
        s = lax.dot_general(a_cur, Mf, (((1,), (0,)), ((), ())),
                            preferred_element_type=jnp.float32)
        return jnp.where(s > 0.0, 0.0, validrow), a_cur, it + 1

    act, _, _ = lax.while_loop(nms_cond, nms_body,
                               (validrow, validrow - 1.0, jnp.int32(0)))

    acti = act.astype(jnp.int32)
    kdest = _prefix_lanes(acti) - 1
    iota128 = lax.broadcasted_iota(jnp.int32, (POST_NMS, PRE_NMS), 0)
    oh2 = ((kdest == iota128) & (acti > 0)).astype(jnp.float32)
    out_ref[0] = _dot_hi(oh2, srt[:, 0:8])


def _const_spec(shape):
    nd = len(shape)
    return pl.BlockSpec(shape, lambda i: (0,) * nd)


def kernel(feature_maps, W1, b1, W2, b2, W3, b3):
    x = feature_maps.reshape((-1, C_IN, FH, FW))
    y = lax.conv_general_dilated(x, W1, (1, 1), [(1, 1), (1, 1)],
                                 dimension_numbers=('NCHW', 'OIHW', 'NCHW'))
    h = jax.nn.relu(y + b1[None, :, None, None]).reshape(L, HID, HW)

    ACX, ACY, AW, AH, IREF, DUMP = _anchor_consts()
    perm = np.array([4 * a_ + k for k in range(4) for a_ in range(A)])
    W2r = W2.reshape(A, HID)
    W3g = W3.reshape(4 * A, HID)[perm]
    b3g = b3[perm]

    head_out_shapes = [jax.ShapeDtypeStruct((L, N_ANC, 16), jnp.float32),
                       jax.ShapeDtypeStruct((L, A, HW), jnp.int32)]
    v16, dest_a = pl.pallas_call(
        _head_body,
        grid=(L,),
        in_specs=[
            pl.BlockSpec((1, HID, HW), lambda i: (i, 0, 0)),
            _const_spec((A, HID)),
            _const_spec((A, 1)),
            _const_spec((4 * A, HID)),
            _const_spec((4 * A, 1)),
            _const_spec((A, HW)),
            _const_spec((A, HW)),
            _const_spec((A, HW)),
            _const_spec((A, HW)),
            _const_spec((A, HW)),
            _const_spec((A, HW)),
            _const_spec((HW, HW)),
            _const_spec((A, A)),
        ],
        out_specs=[pl.BlockSpec((1, N_ANC, 16), lambda i: (i, 0, 0)),
                   pl.BlockSpec((1, A, HW), lambda i: (i, 0, 0))],
        out_shape=head_out_shapes,
        compiler_params=pltpu.CompilerParams(
            dimension_semantics=("parallel",)),
    )(h, W2r, b2[:, None], W3g, b3g[:, None],
      jnp.asarray(ACX), jnp.asarray(ACY), jnp.asarray(AW), jnp.asarray(AH),
      jnp.asarray(DUMP), jnp.asarray(IREF),
      jnp.asarray(np.triu(np.ones((HW, HW), np.float32))),
      jnp.asarray(np.tril(np.ones((A, A), np.float32), -1)))

    vals_t = v16.reshape(N_TILES, ELEM_PER_TILE, 16)
    idx_t = dest_a.reshape(N_TILES, ELEM_PER_TILE // SCAT_BATCH, SCAT_BATCH)

    mesh = plsc.VectorSubcoreMesh(core_axis_name="c", subcore_axis_name="s")
    comp = functools.partial(
        pl.kernel, mesh=mesh,
        compiler_params=pltpu.CompilerParams(use_tc_tiling_on_sc=False),
        out_type=jax.ShapeDtypeStruct((L * ROWS_PER_IMG, 16), jnp.float32),
        scratch_types=[
            pltpu.VMEM((ELEM_PER_TILE, 16), jnp.float32),
            pltpu.VMEM((ELEM_PER_TILE // SCAT_BATCH, SCAT_BATCH), jnp.int32),
            pltpu.SemaphoreType.DMA,
        ],
    )(_sc_route_body)(vals_t, idx_t)

    comp = comp.reshape(L, ROWS_PER_IMG, 16)

    out = pl.pallas_call(
        _tail_body,
        grid=(L,),
        in_specs=[pl.BlockSpec((1, ROWS_PER_IMG, 16), lambda i: (i, 0, 0))],
        out_specs=pl.BlockSpec((1, POST_NMS, 8), lambda i: (i, 0, 0)),
        out_shape=jax.ShapeDtypeStruct((L, POST_NMS, 8), jnp.float32),
        compiler_params=pltpu.CompilerParams(
            dimension_semantics=("parallel",)),
    )(comp)

    return out[:, :, 1:5], out[:, :, 0]
